# baseline (speedup 1.0000x reference)
import functools

import jax
import jax.numpy as jnp
from jax import lax
from jax.experimental import pallas as pl
from jax.experimental.pallas import tpu as pltpu
from jax.experimental.pallas import tpu_sc as plsc

B = 4096
D_IN = 512
D_HID = 1024
D_OUT = 128

BR = 1024
BA = 512
BC = 512
NB_E = B // BR
NB_A = B // BA
NB_J = B // BC

_PREC = lax.Precision.DEFAULT


def _tc_body(xa_ref, xp_ref, w1_ref, b1_ref, w2_ref, b2_ref,
             a_out, p_out, idx_out,
             a_s, pm2_s, an_s, pn_s, dist_s, rmin_s, ridx_s):
    k = pl.program_id(0)

    @pl.when(k < NB_E)
    def _encode():
        w1 = w1_ref[...]
        b1 = b1_ref[...]
        w2 = w2_ref[...]
        b2 = b2_ref[...]
        base = k * BR
        for x_ref, o_ref, is_p in ((xa_ref, a_out, False), (xp_ref, p_out, True)):
            h = jnp.maximum(jnp.dot(x_ref[...], w1, precision=_PREC,
                                    preferred_element_type=jnp.float32) + b1, 0.0)
            o = jnp.dot(h, w2, precision=_PREC,
                        preferred_element_type=jnp.float32) + b2
            o_ref[...] = o
            if is_p:
                pm2_s[pl.ds(base, BR), :] = o * (-2.0)
                pn_s[:, pl.ds(base, BR)] = jnp.sum(o * o, axis=1)[None, :]
            else:
                a_s[pl.ds(base, BR), :] = o
                an_s[pl.ds(base, BR), :] = jnp.sum(o * o, axis=1, keepdims=True)

    @pl.when(k >= NB_E)
    def _argmin():
        km = k - NB_E
        i = km // NB_J
        j = km % NB_J
        ap2 = lax.dot_general(a_s[pl.ds(i * BA, BA), :],
                              pm2_s[pl.ds(j * BC, BC), :],
                              (((1,), (1,)), ((), ())),
                              precision=_PREC, preferred_element_type=jnp.float32)
        dist_s[...] = (an_s[pl.ds(i * BA, BA), :]
                       + pn_s[:, pl.ds(j * BC, BC)]) + ap2

        @pl.when(i == j)
        def _mask():
            d = dist_s[...]
            r = lax.broadcasted_iota(jnp.int32, (BA, BC), 0)
            c = lax.broadcasted_iota(jnp.int32, (BA, BC), 1)
            dist_s[...] = jnp.where(r == c, d + 1e20, d)

        d = dist_s[...]
        m = jnp.min(d, axis=1, keepdims=True)
        wcol = lax.broadcasted_iota(jnp.int32, (BA, BC), 1)
        loc = jnp.min(jnp.where(d == m, wcol, BC), axis=1,
                      keepdims=True)
        gidx = loc + j * BC

        @pl.when(j == 0)
        def _init():
            rmin_s[...] = m
            ridx_s[...] = gidx

        @pl.when(j > 0)
        def _merge():
            better = m < rmin_s[...]
            rmin_s[...] = jnp.where(better, m, rmin_s[...])
            ridx_s[...] = jnp.where(better, gidx, ridx_s[...])

        @pl.when(j == NB_J - 1)
        def _emit():
            idx_out[...] = ridx_s[...]


def _tc_encode_argmin(anchor, pos, W1, b1, W2, b2):
    return pl.pallas_call(
        _tc_body,
        grid=(NB_E + NB_A * NB_J,),
        in_specs=[
            pl.BlockSpec((BR, D_IN), lambda k: (jnp.minimum(k, NB_E - 1), 0)),
            pl.BlockSpec((BR, D_IN), lambda k: (jnp.minimum(k, NB_E - 1), 0)),
            pl.BlockSpec((D_IN, D_HID), lambda k: (0, 0)),
            pl.BlockSpec((1, D_HID), lambda k: (0, 0)),
            pl.BlockSpec((D_HID, D_OUT), lambda k: (0, 0)),
            pl.BlockSpec((1, D_OUT), lambda k: (0, 0)),
        ],
        out_specs=[
            pl.BlockSpec((BR, D_OUT), lambda k: (jnp.minimum(k, NB_E - 1), 0)),
            pl.BlockSpec((BR, D_OUT), lambda k: (jnp.minimum(k, NB_E - 1), 0)),
            pl.BlockSpec((BA, 1), lambda k: (jnp.maximum(k - NB_E, 0) // NB_J, 0)),
        ],
        out_shape=[
            jax.ShapeDtypeStruct((B, D_OUT), jnp.float32),
            jax.ShapeDtypeStruct((B, D_OUT), jnp.float32),
            jax.ShapeDtypeStruct((B, 1), jnp.int32),
        ],
        scratch_shapes=[
            pltpu.VMEM((B, D_OUT), jnp.float32),
            pltpu.VMEM((B, D_OUT), jnp.float32),
            pltpu.VMEM((B, 1), jnp.float32),
            pltpu.VMEM((1, B), jnp.float32),
            pltpu.VMEM((BA, BC), jnp.float32),
            pltpu.VMEM((BA, 1), jnp.float32),
            pltpu.VMEM((BA, 1), jnp.int32),
        ],
    )(anchor, pos, W1, b1.reshape(1, D_HID), W2, b2.reshape(1, D_OUT))


def _sc_finalize(a, p, idx):
    info = plsc.get_sparse_core_info()
    nc, ns = info.num_cores, info.num_subcores
    nw = nc * ns
    bw = B // nw
    mesh = plsc.VectorSubcoreMesh(core_axis_name="c", subcore_axis_name="s")

    @functools.partial(
        pl.kernel,
        mesh=mesh,
        out_type=jax.ShapeDtypeStruct((B, 3 * D_OUT), jnp.float32),
        scratch_types=[
            pltpu.VMEM((bw,), jnp.int32),
            pltpu.VMEM((bw, 3 * D_OUT), jnp.float32),
            pltpu.SemaphoreType.DMA,
            pltpu.SemaphoreType.DMA,
            pltpu.SemaphoreType.DMA,
            pltpu.SemaphoreType.DMA,
        ],
    )
    def finalize_k(a_hbm, p_hbm, idx_hbm, out_hbm, idx_v, tile_v,
                   sem_i, sem_a, sem_p, sem_g):
        wid = lax.axis_index("s") * nc + lax.axis_index("c")
        base = wid * bw
        ci = pltpu.async_copy(idx_hbm.at[pl.ds(base, bw)], idx_v, sem_i)
        ca = pltpu.async_copy(a_hbm.at[pl.ds(base, bw)],
                              tile_v.at[:, pl.ds(0, D_OUT)], sem_a)
        cp = pltpu.async_copy(p_hbm.at[pl.ds(base, bw)],
                              tile_v.at[:, pl.ds(D_OUT, D_OUT)], sem_p)
        ci.wait()
        cg = pltpu.async_copy(p_hbm.at[idx_v],
                              tile_v.at[:, pl.ds(2 * D_OUT, D_OUT)], sem_g)
        ca.wait()
        cp.wait()
        cg.wait()
        pltpu.sync_copy(tile_v, out_hbm.at[pl.ds(base, bw)])

    return finalize_k(a, p, idx)


def kernel(anchor, pos, W1, b1, W2, b2):
    a, p, idx = _tc_encode_argmin(anchor, pos, W1, b1, W2, b2)
    return _sc_finalize(a, p, idx.reshape(B))


# speedup vs baseline: 1.4421x; 1.4421x over previous
import functools

import jax
import jax.numpy as jnp
from jax import lax
from jax.experimental import pallas as pl
from jax.experimental.pallas import tpu as pltpu
from jax.experimental.pallas import tpu_sc as plsc

B = 4096
D_IN = 512
D_HID = 1024
D_OUT = 128

BR = 1024
BA = 512
BC = 512
NB_E = B // BR
NB_A = B // BA
NB_J = B // BC

_PREC = lax.Precision.DEFAULT


def _tc_body(xa_ref, xp_ref, w1_ref, b1_ref, w2_ref, b2_ref,
             a_out, p_out, idx_out, a_s, pm2_s, pn_s):
    k = pl.program_id(0)

    @pl.when(k < NB_E)
    def _encode():
        w1 = w1_ref[...]
        b1 = b1_ref[...]
        w2 = w2_ref[...]
        b2 = b2_ref[...]
        base = k * BR
        for x_ref, o_ref, is_p in ((xa_ref, a_out, False), (xp_ref, p_out, True)):
            h = jnp.maximum(jnp.dot(x_ref[...], w1, precision=_PREC,
                                    preferred_element_type=jnp.float32) + b1, 0.0)
            o = jnp.dot(h, w2, precision=_PREC,
                        preferred_element_type=jnp.float32) + b2
            o_ref[...] = o
            if is_p:
                pm2_s[pl.ds(base, BR), :] = o * (-2.0)
                pn_s[:, pl.ds(base, BR)] = jnp.sum(o * o, axis=1)[None, :]
            else:
                a_s[pl.ds(base, BR), :] = o

    @pl.when(k >= NB_E)
    def _argmin():
        i = k - NB_E
        a = a_s[pl.ds(i * BA, BA), :]
        an = jnp.sum(a * a, axis=1, keepdims=True)
        ap2 = lax.dot_general(a, pm2_s[...], (((1,), (1,)), ((), ())),
                              precision=_PREC, preferred_element_type=jnp.float32)
        dist = (an + pn_s[...]) + ap2
        rows = i * BA + lax.broadcasted_iota(jnp.int32, (BA, B), 0)
        cols = lax.broadcasted_iota(jnp.int32, (BA, B), 1)
        dist = jnp.where(rows == cols, dist + 1e20, dist)
        idx_out[0, 0, :] = jnp.argmin(dist, axis=1).astype(jnp.int32)


def _tc_encode_argmin(anchor, pos, W1, b1, W2, b2):
    return pl.pallas_call(
        _tc_body,
        grid=(NB_E + NB_A,),
        in_specs=[
            pl.BlockSpec((BR, D_IN), lambda k: (jnp.minimum(k, NB_E - 1), 0)),
            pl.BlockSpec((BR, D_IN), lambda k: (jnp.minimum(k, NB_E - 1), 0)),
            pl.BlockSpec((D_IN, D_HID), lambda k: (0, 0)),
            pl.BlockSpec((1, D_HID), lambda k: (0, 0)),
            pl.BlockSpec((D_HID, D_OUT), lambda k: (0, 0)),
            pl.BlockSpec((1, D_OUT), lambda k: (0, 0)),
        ],
        out_specs=[
            pl.BlockSpec((BR, D_OUT), lambda k: (jnp.minimum(k, NB_E - 1), 0)),
            pl.BlockSpec((BR, D_OUT), lambda k: (jnp.minimum(k, NB_E - 1), 0)),
            pl.BlockSpec((1, 1, BA), lambda k: (jnp.maximum(k - NB_E, 0), 0, 0)),
        ],
        out_shape=[
            jax.ShapeDtypeStruct((B, D_OUT), jnp.float32),
            jax.ShapeDtypeStruct((B, D_OUT), jnp.float32),
            jax.ShapeDtypeStruct((NB_A, 1, BA), jnp.int32),
        ],
        scratch_shapes=[
            pltpu.VMEM((B, D_OUT), jnp.float32),
            pltpu.VMEM((B, D_OUT), jnp.float32),
            pltpu.VMEM((1, B), jnp.float32),
        ],
    )(anchor, pos, W1, b1.reshape(1, D_HID), W2, b2.reshape(1, D_OUT))


def _sc_finalize(a, p, idx):
    info = plsc.get_sparse_core_info()
    nc, ns = info.num_cores, info.num_subcores
    nw = nc * ns
    bw = B // nw
    mesh = plsc.VectorSubcoreMesh(core_axis_name="c", subcore_axis_name="s")

    @functools.partial(
        pl.kernel,
        mesh=mesh,
        out_type=jax.ShapeDtypeStruct((B, 3 * D_OUT), jnp.float32),
        scratch_types=[
            pltpu.VMEM((bw,), jnp.int32),
            pltpu.VMEM((bw, 3 * D_OUT), jnp.float32),
            pltpu.SemaphoreType.DMA,
            pltpu.SemaphoreType.DMA,
            pltpu.SemaphoreType.DMA,
            pltpu.SemaphoreType.DMA,
        ],
    )
    def finalize_k(a_hbm, p_hbm, idx_hbm, out_hbm, idx_v, tile_v,
                   sem_i, sem_a, sem_p, sem_g):
        wid = lax.axis_index("s") * nc + lax.axis_index("c")
        base = wid * bw
        ci = pltpu.async_copy(idx_hbm.at[pl.ds(base, bw)], idx_v, sem_i)
        ca = pltpu.async_copy(a_hbm.at[pl.ds(base, bw)],
                              tile_v.at[:, pl.ds(0, D_OUT)], sem_a)
        cp = pltpu.async_copy(p_hbm.at[pl.ds(base, bw)],
                              tile_v.at[:, pl.ds(D_OUT, D_OUT)], sem_p)
        ci.wait()
        cg = pltpu.async_copy(p_hbm.at[idx_v],
                              tile_v.at[:, pl.ds(2 * D_OUT, D_OUT)], sem_g)
        ca.wait()
        cp.wait()
        cg.wait()
        pltpu.sync_copy(tile_v, out_hbm.at[pl.ds(base, bw)])

    return finalize_k(a, p, idx)


def kernel(anchor, pos, W1, b1, W2, b2):
    a, p, idx = _tc_encode_argmin(anchor, pos, W1, b1, W2, b2)
    return _sc_finalize(a, p, idx.reshape(B))


# speedup vs baseline: 1.5670x; 1.0866x over previous
import functools

import jax
import jax.numpy as jnp
from jax import lax
from jax.experimental import pallas as pl
from jax.experimental.pallas import tpu as pltpu
from jax.experimental.pallas import tpu_sc as plsc

B = 4096
D_IN = 512
D_HID = 1024
D_OUT = 128

BR = 1024
BA = 512
BC = 512
NB_E = B // BR
NB_A = B // BA
NB_J = B // BC

_PREC = lax.Precision.DEFAULT


def _tc_body(xa_ref, xp_ref, w1_ref, b1_ref, w2_ref, b2_ref,
             a_out, p_out, idx_out, a_s, pm2_s, pn_s):
    k = pl.program_id(0)

    @pl.when(k < NB_E)
    def _encode():
        w1 = w1_ref[...]
        b1 = b1_ref[...]
        w2 = w2_ref[...]
        b2 = b2_ref[...]
        base = k * BR
        for x_ref, o_ref, is_p in ((xa_ref, a_out, False), (xp_ref, p_out, True)):
            h = jnp.maximum(jnp.dot(x_ref[...], w1, precision=_PREC,
                                    preferred_element_type=jnp.float32) + b1, 0.0)
            o = jnp.dot(h, w2, precision=_PREC,
                        preferred_element_type=jnp.float32) + b2
            o_ref[...] = o
            if is_p:
                pm2_s[pl.ds(base, BR), :] = o * (-2.0)
                pn_s[:, pl.ds(base, BR)] = jnp.sum(o * o, axis=1)[None, :]
            else:
                a_s[pl.ds(base, BR), :] = o

    @pl.when(k >= NB_E)
    def _argmin():
        i = k - NB_E
        a = a_s[pl.ds(i * BA, BA), :]
        an = jnp.sum(a * a, axis=1, keepdims=True)
        ap2 = lax.dot_general(a, pm2_s[...], (((1,), (1,)), ((), ())),
                              precision=_PREC, preferred_element_type=jnp.float32)
        base = (an + pn_s[...]) + ap2
        eye20 = jnp.where(
            lax.broadcasted_iota(jnp.int32, (BA, BA), 0)
            == lax.broadcasted_iota(jnp.int32, (BA, BA), 1),
            jnp.float32(1e20), jnp.float32(0.0))
        chunks = []
        for c in range(B // BA):
            ch = lax.slice_in_dim(base, c * BA, (c + 1) * BA, axis=1)
            chunks.append(jnp.where(i == c, ch + eye20, ch))
        dist = jnp.concatenate(chunks, axis=1)
        idx_out[0, 0, :] = jnp.argmin(dist, axis=1).astype(jnp.int32)


def _tc_encode_argmin(anchor, pos, W1, b1, W2, b2):
    return pl.pallas_call(
        _tc_body,
        grid=(NB_E + NB_A,),
        in_specs=[
            pl.BlockSpec((BR, D_IN), lambda k: (jnp.minimum(k, NB_E - 1), 0)),
            pl.BlockSpec((BR, D_IN), lambda k: (jnp.minimum(k, NB_E - 1), 0)),
            pl.BlockSpec((D_IN, D_HID), lambda k: (0, 0)),
            pl.BlockSpec((1, D_HID), lambda k: (0, 0)),
            pl.BlockSpec((D_HID, D_OUT), lambda k: (0, 0)),
            pl.BlockSpec((1, D_OUT), lambda k: (0, 0)),
        ],
        out_specs=[
            pl.BlockSpec((BR, D_OUT), lambda k: (jnp.minimum(k, NB_E - 1), 0)),
            pl.BlockSpec((BR, D_OUT), lambda k: (jnp.minimum(k, NB_E - 1), 0)),
            pl.BlockSpec((1, 1, BA), lambda k: (jnp.maximum(k - NB_E, 0), 0, 0)),
        ],
        out_shape=[
            jax.ShapeDtypeStruct((B, D_OUT), jnp.float32),
            jax.ShapeDtypeStruct((B, D_OUT), jnp.float32),
            jax.ShapeDtypeStruct((NB_A, 1, BA), jnp.int32),
        ],
        scratch_shapes=[
            pltpu.VMEM((B, D_OUT), jnp.float32),
            pltpu.VMEM((B, D_OUT), jnp.float32),
            pltpu.VMEM((1, B), jnp.float32),
        ],
    )(anchor, pos, W1, b1.reshape(1, D_HID), W2, b2.reshape(1, D_OUT))


def _sc_finalize(a, p, idx):
    info = plsc.get_sparse_core_info()
    nc, ns = info.num_cores, info.num_subcores
    nw = nc * ns
    bw = B // nw
    mesh = plsc.VectorSubcoreMesh(core_axis_name="c", subcore_axis_name="s")

    @functools.partial(
        pl.kernel,
        mesh=mesh,
        out_type=jax.ShapeDtypeStruct((B, 3 * D_OUT), jnp.float32),
        scratch_types=[
            pltpu.VMEM((bw,), jnp.int32),
            pltpu.VMEM((bw, 3 * D_OUT), jnp.float32),
            pltpu.SemaphoreType.DMA,
            pltpu.SemaphoreType.DMA,
            pltpu.SemaphoreType.DMA,
            pltpu.SemaphoreType.DMA,
        ],
    )
    def finalize_k(a_hbm, p_hbm, idx_hbm, out_hbm, idx_v, tile_v,
                   sem_i, sem_a, sem_p, sem_g):
        wid = lax.axis_index("s") * nc + lax.axis_index("c")
        base = wid * bw
        ci = pltpu.async_copy(idx_hbm.at[pl.ds(base, bw)], idx_v, sem_i)
        ca = pltpu.async_copy(a_hbm.at[pl.ds(base, bw)],
                              tile_v.at[:, pl.ds(0, D_OUT)], sem_a)
        cp = pltpu.async_copy(p_hbm.at[pl.ds(base, bw)],
                              tile_v.at[:, pl.ds(D_OUT, D_OUT)], sem_p)
        ci.wait()
        cg = pltpu.async_copy(p_hbm.at[idx_v],
                              tile_v.at[:, pl.ds(2 * D_OUT, D_OUT)], sem_g)
        ca.wait()
        cp.wait()
        cg.wait()
        pltpu.sync_copy(tile_v, out_hbm.at[pl.ds(base, bw)])

    return finalize_k(a, p, idx)


def kernel(anchor, pos, W1, b1, W2, b2):
    a, p, idx = _tc_encode_argmin(anchor, pos, W1, b1, W2, b2)
    return _sc_finalize(a, p, idx.reshape(B))
